# baseline (speedup 1.0000x reference)
import jax
import jax.numpy as jnp
from jax.experimental import pallas as pl
from jax.experimental.pallas import tpu as pltpu
from jax.experimental.pallas import tpu_sc as plsc

_C = 32768
_G = 256
_H = 128
_P = 0.9
_GB = 8
_CK = 2048
_NW = 32
_NV = _C // 16
_MININT = -2**31


def _z_body(nact_ref, w1_ref, b1_ref, w2b_ref, b2_ref,
            et_ref, nt_ref, z_ref):
    nact = nact_ref[0]
    b2 = b2_ref[0]

    def chunk_step(c, _):
        sl = pl.ds(c * _CK, _CK)
        x = et_ref[:, sl]

        def jstep(j, acc):
            h = jnp.maximum(x * w1_ref[j] + b1_ref[j], 0.0)
            hb = h.astype(jnp.bfloat16).astype(jnp.float32)
            return acc + w2b_ref[j] * hb

        logits = jax.lax.fori_loop(0, nact, jstep,
                                   jnp.zeros((_GB, _CK), jnp.float32))
        z_ref[:, sl] = (logits + b2) + nt_ref[:, sl]
        return 0

    jax.lax.fori_loop(0, _C // _CK, chunk_step, 0)


def _uk(zvec):
    u = jax.lax.bitcast_convert_type(zvec, jnp.int32)
    return jnp.where(u < 0, ~u, u | jnp.int32(_MININT))


def _zero_hist(hist_ref, nchunks):
    def zstep(i, _):
        hist_ref[pl.ds(i * 16, 16)] = jnp.zeros((16,), jnp.float32)
        return 0
    jax.lax.fori_loop(0, nchunks, zstep, 0)


def _scal(x):
    return jnp.max(x) if getattr(x, "ndim", 0) else x


def _cross_scan(hist_refs, nchunks, a0, budget):
    iota = jax.lax.iota(jnp.int32, 16)

    def step(t, st):
        arun, found, bb, across = st
        chunk = nchunks - 1 - t
        hv = hist_refs[0][pl.ds(chunk * 16, 16)]
        for hr in hist_refs[1:]:
            hv = hv + hr[pl.ds(chunk * 16, 16)]
        rv = jax.lax.rev(hv, (0,))
        cs = plsc.cumsum(rv)
        cum = arun + cs
        crossed = cum > budget
        npos = _scal(plsc.all_reduce_population_count(crossed))
        f = _scal(plsc.all_reduce_ffs(crossed))
        has = npos > 0
        a_cand = arun + jnp.sum(jnp.where(iota == f - 1, cs, 0.0))
        b_cand = chunk * 16 + (15 - f)
        chunk_sum = jnp.sum(jnp.where(iota == 15, cs, 0.0))
        take = has & (found == 0)
        bb = jnp.where(take, b_cand, bb)
        across = jnp.where(take, a_cand, across)
        found = jnp.where(has, 1, found)
        return arun + chunk_sum, found, bb, across

    st = jax.lax.fori_loop(0, nchunks, step,
                           (a0, jnp.int32(0), jnp.int32(0), a0))
    return st[1], st[2], st[3]


def _sel_body(z_hbm, out_hbm, zv, ev, kv,
              h0, h1, h2, h3, c0, c1, c2, c3, mt_ref):
    wid = jax.lax.axis_index("s") * 2 + jax.lax.axis_index("c")
    iota = jax.lax.iota(jnp.int32, 16)

    def wave_body(w, _):
        g = w * _NW + wid
        pltpu.sync_copy(z_hbm.at[g], zv)

        hbanks = (h0, h1, h2, h3)
        cbanks = (c0, c1, c2, c3)
        for hb in hbanks:
            _zero_hist(hb, 16)

        @plsc.parallel_loop(0, _NV // 4, 1, unroll=4,
                            carry=jnp.zeros((16,), jnp.float32))
        def p2(i, tacc):
            for u in range(4):
                sl = pl.ds((i * 4 + u) * 16, 16)
                evec = jnp.exp(zv[sl])
                ev[sl] = evec
                ku = _uk(zv[sl])
                kv[sl] = ku
                b = jax.lax.shift_right_logical(ku, 24)
                plsc.addupdate_scatter(hbanks[u], [b], evec)
                tacc = tacc + evec
            return tacc

        tvec = p2
        budget = jnp.float32(_P) * jnp.sum(tvec)

        _, b1_, a1 = _cross_scan(hbanks, 16, jnp.float32(0.0), budget)

        def key_level(shift, prefix, a_prev, with_cnt):
            for hb in hbanks:
                _zero_hist(hb, 16)
            if with_cnt:
                def zc(i, _):
                    for cb in cbanks:
                        cb[pl.ds(i * 16, 16)] = jnp.zeros((16,), jnp.int32)
                    return 0
                jax.lax.fori_loop(0, 16, zc, 0)

            @plsc.parallel_loop(0, _NV // 4, 1, unroll=4)
            def body(i):
                for u in range(4):
                    sl = pl.ds((i * 4 + u) * 16, 16)
                    ku = kv[sl]
                    pmask = jax.lax.shift_right_logical(ku, shift + 8) == prefix
                    b = jax.lax.shift_right_logical(ku, shift) & 255
                    plsc.addupdate_scatter(hbanks[u], [b], ev[sl], mask=pmask)
                    if with_cnt:
                        plsc.addupdate_scatter(
                            cbanks[u], [b], jnp.ones((16,), jnp.int32),
                            mask=pmask)
            _, bb, aa = _cross_scan(hbanks, 16, a_prev, budget)
            return bb, aa

        b2_, a2 = key_level(16, b1_, a1, False)
        b3_, a3 = key_level(8, (b1_ << 8) | b2_, a2, False)
        prefix24 = ((b1_ << 8) | b2_) << 8 | b3_
        b4_, a4 = key_level(0, prefix24, a3, True)
        vk = (prefix24 << 8) | b4_

        cv = (c0[pl.ds((b4_ >> 4) * 16, 16)] + c1[pl.ds((b4_ >> 4) * 16, 16)]
              + c2[pl.ds((b4_ >> 4) * 16, 16)] + c3[pl.ds((b4_ >> 4) * 16, 16)])
        nt = jnp.sum(jnp.where(iota == (b4_ & 15), cv, 0))

        mt_ref[...] = jnp.zeros((16,), jnp.int32) + jnp.where(
            a4 == 0.0, jnp.int32(-1), jnp.int32(_C))

        @pl.when(nt > 1)
        def _idx_levels():
            _zero_hist(h0, 16)

            def b5body(i, _):
                sl = pl.ds(i * 16, 16)
                minor = (_C - 1) - (i * 16 + iota)
                b = jax.lax.shift_right_logical(minor, 7)
                plsc.addupdate_scatter(h0, [b], ev[sl], mask=(kv[sl] == vk))
                return 0

            jax.lax.fori_loop(0, _NV, b5body, 0)
            f5, b5_, a5 = _cross_scan((h0,), 16, a4, budget)

            _zero_hist(h0, 8)

            def b6body(i, _):
                sl = pl.ds(i * 16, 16)
                minor = (_C - 1) - (i * 16 + iota)
                pm = (kv[sl] == vk) & (
                    jax.lax.shift_right_logical(minor, 7) == b5_)
                plsc.addupdate_scatter(h0, [minor & 127], ev[sl], mask=pm)
                return 0

            jax.lax.fori_loop(0, _NV, b6body, 0)
            f6, b6_, a6 = _cross_scan((h0,), 8, a5, budget)

            mt = (b5_ << 7) | b6_
            mt = jnp.where(a6 == 0.0, mt - 1, mt)
            mt = jnp.where(f6 == 0, jnp.int32(-1), mt)
            mt_ref[...] = jnp.zeros((16,), jnp.int32) + mt

        vks = vk ^ jnp.int32(_MININT)
        mt_eff = mt_ref[...]

        @plsc.parallel_loop(0, _NV // 4, 1, unroll=4)
        def pf(i):
            for u in range(4):
                ii = i * 4 + u
                sl = pl.ds(ii * 16, 16)
                ku = kv[sl]
                minor = (_C - 1) - (ii * 16 + iota)
                sel = ((ku ^ jnp.int32(_MININT)) > vks) | (
                    (ku == vk) & (minor > mt_eff))
                ev[sl] = jnp.where(sel, 1.0, 0.0)
        pltpu.sync_copy(ev, out_hbm.at[g])
        return 0

    jax.lax.fori_loop(0, z_hbm.shape[0] // _NW, wave_body, 0)


def kernel(expression_matrix, W1, b1, W2, b2, gumbel_noise):
    w1 = W1[:, 0]
    w2 = W2[0]

    on = jnp.maximum(b1, w1 + b1) > 0.0
    order = jnp.argsort(~on)
    w1p = w1[order]
    b1p = b1[order]
    w2bp = w2[order].astype(jnp.bfloat16).astype(jnp.float32)
    nact = jnp.sum(on.astype(jnp.int32)).reshape((1,))

    h_hi = jnp.maximum(jnp.maximum(b1, w1 + b1), 0.0)
    h_lo = jnp.maximum(jnp.minimum(b1, w1 + b1), 0.0)
    z_ub = jnp.sum(jnp.where(w2bp > 0, w2bp * h_hi[order], w2bp * h_lo[order]))
    b2_adj = b2 - (z_ub + 14.0)

    row_spec = pl.BlockSpec((_GB, _C), lambda i: (i, 0))
    smem = pl.BlockSpec(memory_space=pltpu.SMEM)
    gh = _G // 8

    def z_half(eth, nth):
        return pl.pallas_call(
            _z_body,
            grid=(gh // _GB,),
            in_specs=[smem, smem, smem, smem, smem, row_spec, row_spec],
            out_specs=row_spec,
            out_shape=jax.ShapeDtypeStruct((gh, _C), jnp.float32),
        )(nact, w1p, b1p, w2bp, b2_adj, eth, nth)

    sel = pl.kernel(
        _sel_body,
        out_type=jax.ShapeDtypeStruct((gh, _C), jnp.float32),
        mesh=plsc.VectorSubcoreMesh(core_axis_name="c", subcore_axis_name="s"),
        scratch_types=[
            pltpu.VMEM((_C,), jnp.float32),
            pltpu.VMEM((_C,), jnp.float32),
            pltpu.VMEM((_C,), jnp.int32),
            pltpu.VMEM((256,), jnp.float32),
            pltpu.VMEM((256,), jnp.float32),
            pltpu.VMEM((256,), jnp.float32),
            pltpu.VMEM((256,), jnp.float32),
            pltpu.VMEM((256,), jnp.int32),
            pltpu.VMEM((256,), jnp.int32),
            pltpu.VMEM((256,), jnp.int32),
            pltpu.VMEM((256,), jnp.int32),
            pltpu.VMEM((16,), jnp.int32),
        ],
        compiler_params=pltpu.CompilerParams(needs_layout_passes=False),
    )
    zs = [z_half(expression_matrix[:, k * gh:(k + 1) * gh].T,
                 gumbel_noise[:, k * gh:(k + 1) * gh].T)
          for k in range(8)]
    hs = [sel(z) for z in zs]
    ht = jnp.concatenate(hs, axis=0)
    return ht.T


# speedup vs baseline: 1.3117x; 1.3117x over previous
import jax
import jax.numpy as jnp
from jax.experimental import pallas as pl
from jax.experimental.pallas import tpu as pltpu
from jax.experimental.pallas import tpu_sc as plsc

_C = 32768
_G = 256
_H = 128
_P = 0.9
_GB = 16
_CK = 2048
_NW = 32
_NV = _C // 16
_MININT = -2**31


def _z_body(nact_ref, w1_ref, b1_ref, w2b_ref, b2_ref,
            et_ref, nt_ref, z_ref):
    nact = nact_ref[0]
    b2 = b2_ref[0]

    def chunk_step(c, _):
        sl = pl.ds(c * _CK, _CK)
        x = et_ref[:, sl]

        def jstep(j, acc):
            h = jnp.maximum(x * w1_ref[j] + b1_ref[j], 0.0)
            hb = h.astype(jnp.bfloat16).astype(jnp.float32)
            return acc + w2b_ref[j] * hb

        logits = jax.lax.fori_loop(0, nact, jstep,
                                   jnp.zeros((_GB, _CK), jnp.float32))
        z_ref[:, sl] = (logits + b2) + nt_ref[:, sl]
        return 0

    jax.lax.fori_loop(0, _C // _CK, chunk_step, 0)


def _uk(zvec):
    u = jax.lax.bitcast_convert_type(zvec, jnp.int32)
    return jnp.where(u < 0, ~u, u | jnp.int32(_MININT))


def _zero_hist(hist_ref, nchunks):
    def zstep(i, _):
        hist_ref[pl.ds(i * 16, 16)] = jnp.zeros((16,), jnp.float32)
        return 0
    jax.lax.fori_loop(0, nchunks, zstep, 0)


def _scal(x):
    return jnp.max(x) if getattr(x, "ndim", 0) else x


def _cross_scan(hist_refs, nchunks, a0, budget):
    iota = jax.lax.iota(jnp.int32, 16)

    def step(t, st):
        arun, found, bb, across = st
        chunk = nchunks - 1 - t
        hv = hist_refs[0][pl.ds(chunk * 16, 16)]
        for hr in hist_refs[1:]:
            hv = hv + hr[pl.ds(chunk * 16, 16)]
        rv = jax.lax.rev(hv, (0,))
        cs = plsc.cumsum(rv)
        cum = arun + cs
        crossed = cum > budget
        npos = _scal(plsc.all_reduce_population_count(crossed))
        f = _scal(plsc.all_reduce_ffs(crossed))
        has = npos > 0
        a_cand = arun + jnp.sum(jnp.where(iota == f - 1, cs, 0.0))
        b_cand = chunk * 16 + (15 - f)
        chunk_sum = jnp.sum(jnp.where(iota == 15, cs, 0.0))
        take = has & (found == 0)
        bb = jnp.where(take, b_cand, bb)
        across = jnp.where(take, a_cand, across)
        found = jnp.where(has, 1, found)
        return arun + chunk_sum, found, bb, across

    st = jax.lax.fori_loop(0, nchunks, step,
                           (a0, jnp.int32(0), jnp.int32(0), a0))
    return st[1], st[2], st[3]


def _sel_body(z_hbm, out_hbm, zv, ev, kv,
              h0, h1, h2, h3, c0, c1, c2, c3, mt_ref):
    wid = jax.lax.axis_index("s") * 2 + jax.lax.axis_index("c")
    iota = jax.lax.iota(jnp.int32, 16)

    def wave_body(w, _):
        g = w * _NW + wid
        pltpu.sync_copy(z_hbm.at[g], zv)

        hbanks = (h0, h1, h2, h3)
        cbanks = (c0, c1, c2, c3)
        for hb in hbanks:
            _zero_hist(hb, 16)

        @plsc.parallel_loop(0, _NV // 4, 1, unroll=4,
                            carry=jnp.zeros((16,), jnp.float32))
        def p2(i, tacc):
            for u in range(4):
                sl = pl.ds((i * 4 + u) * 16, 16)
                evec = jnp.exp(zv[sl])
                ev[sl] = evec
                ku = _uk(zv[sl])
                kv[sl] = ku
                b = jax.lax.shift_right_logical(ku, 24)
                plsc.addupdate_scatter(hbanks[u], [b], evec)
                tacc = tacc + evec
            return tacc

        tvec = p2
        budget = jnp.float32(_P) * jnp.sum(tvec)

        _, b1_, a1 = _cross_scan(hbanks, 16, jnp.float32(0.0), budget)

        def key_level(shift, prefix, a_prev, with_cnt):
            for hb in hbanks:
                _zero_hist(hb, 16)
            if with_cnt:
                def zc(i, _):
                    for cb in cbanks:
                        cb[pl.ds(i * 16, 16)] = jnp.zeros((16,), jnp.int32)
                    return 0
                jax.lax.fori_loop(0, 16, zc, 0)

            @plsc.parallel_loop(0, _NV // 4, 1, unroll=4)
            def body(i):
                for u in range(4):
                    sl = pl.ds((i * 4 + u) * 16, 16)
                    ku = kv[sl]
                    pmask = jax.lax.shift_right_logical(ku, shift + 8) == prefix
                    b = jax.lax.shift_right_logical(ku, shift) & 255
                    plsc.addupdate_scatter(hbanks[u], [b], ev[sl], mask=pmask)
                    if with_cnt:
                        plsc.addupdate_scatter(
                            cbanks[u], [b], jnp.ones((16,), jnp.int32),
                            mask=pmask)
            _, bb, aa = _cross_scan(hbanks, 16, a_prev, budget)
            return bb, aa

        b2_, a2 = key_level(16, b1_, a1, False)
        b3_, a3 = key_level(8, (b1_ << 8) | b2_, a2, False)
        prefix24 = ((b1_ << 8) | b2_) << 8 | b3_
        b4_, a4 = key_level(0, prefix24, a3, True)
        vk = (prefix24 << 8) | b4_

        cv = (c0[pl.ds((b4_ >> 4) * 16, 16)] + c1[pl.ds((b4_ >> 4) * 16, 16)]
              + c2[pl.ds((b4_ >> 4) * 16, 16)] + c3[pl.ds((b4_ >> 4) * 16, 16)])
        nt = jnp.sum(jnp.where(iota == (b4_ & 15), cv, 0))

        mt_ref[...] = jnp.zeros((16,), jnp.int32) + jnp.where(
            a4 == 0.0, jnp.int32(-1), jnp.int32(_C))

        @pl.when(nt > 1)
        def _idx_levels():
            _zero_hist(h0, 16)

            def b5body(i, _):
                sl = pl.ds(i * 16, 16)
                minor = (_C - 1) - (i * 16 + iota)
                b = jax.lax.shift_right_logical(minor, 7)
                plsc.addupdate_scatter(h0, [b], ev[sl], mask=(kv[sl] == vk))
                return 0

            jax.lax.fori_loop(0, _NV, b5body, 0)
            f5, b5_, a5 = _cross_scan((h0,), 16, a4, budget)

            _zero_hist(h0, 8)

            def b6body(i, _):
                sl = pl.ds(i * 16, 16)
                minor = (_C - 1) - (i * 16 + iota)
                pm = (kv[sl] == vk) & (
                    jax.lax.shift_right_logical(minor, 7) == b5_)
                plsc.addupdate_scatter(h0, [minor & 127], ev[sl], mask=pm)
                return 0

            jax.lax.fori_loop(0, _NV, b6body, 0)
            f6, b6_, a6 = _cross_scan((h0,), 8, a5, budget)

            mt = (b5_ << 7) | b6_
            mt = jnp.where(a6 == 0.0, mt - 1, mt)
            mt = jnp.where(f6 == 0, jnp.int32(-1), mt)
            mt_ref[...] = jnp.zeros((16,), jnp.int32) + mt

        vks = vk ^ jnp.int32(_MININT)
        mt_eff = mt_ref[...]

        @plsc.parallel_loop(0, _NV // 4, 1, unroll=4)
        def pf(i):
            for u in range(4):
                ii = i * 4 + u
                sl = pl.ds(ii * 16, 16)
                ku = kv[sl]
                minor = (_C - 1) - (ii * 16 + iota)
                sel = ((ku ^ jnp.int32(_MININT)) > vks) | (
                    (ku == vk) & (minor > mt_eff))
                ev[sl] = jnp.where(sel, 1.0, 0.0)
        pltpu.sync_copy(ev, out_hbm.at[g])
        return 0

    jax.lax.fori_loop(0, z_hbm.shape[0] // _NW, wave_body, 0)


def kernel(expression_matrix, W1, b1, W2, b2, gumbel_noise):
    et = expression_matrix.T
    nt = gumbel_noise.T
    w1 = W1[:, 0]
    w2 = W2[0]

    on = jnp.maximum(b1, w1 + b1) > 0.0
    order = jnp.argsort(~on)
    w1p = w1[order]
    b1p = b1[order]
    w2bp = w2[order].astype(jnp.bfloat16).astype(jnp.float32)
    nact = jnp.sum(on.astype(jnp.int32)).reshape((1,))

    h_hi = jnp.maximum(jnp.maximum(b1, w1 + b1), 0.0)
    h_lo = jnp.maximum(jnp.minimum(b1, w1 + b1), 0.0)
    z_ub = jnp.sum(jnp.where(w2bp > 0, w2bp * h_hi[order], w2bp * h_lo[order]))
    b2_adj = b2 - (z_ub + 14.0)

    row_spec = pl.BlockSpec((_GB, _C), lambda i: (i, 0))
    smem = pl.BlockSpec(memory_space=pltpu.SMEM)
    gh = _G // 8

    def z_half(eth, nth):
        return pl.pallas_call(
            _z_body,
            grid=(gh // _GB,),
            in_specs=[smem, smem, smem, smem, smem, row_spec, row_spec],
            out_specs=row_spec,
            out_shape=jax.ShapeDtypeStruct((gh, _C), jnp.float32),
        )(nact, w1p, b1p, w2bp, b2_adj, eth, nth)

    sel = pl.kernel(
        _sel_body,
        out_type=jax.ShapeDtypeStruct((gh, _C), jnp.float32),
        mesh=plsc.VectorSubcoreMesh(core_axis_name="c", subcore_axis_name="s"),
        scratch_types=[
            pltpu.VMEM((_C,), jnp.float32),
            pltpu.VMEM((_C,), jnp.float32),
            pltpu.VMEM((_C,), jnp.int32),
            pltpu.VMEM((256,), jnp.float32),
            pltpu.VMEM((256,), jnp.float32),
            pltpu.VMEM((256,), jnp.float32),
            pltpu.VMEM((256,), jnp.float32),
            pltpu.VMEM((256,), jnp.int32),
            pltpu.VMEM((256,), jnp.int32),
            pltpu.VMEM((256,), jnp.int32),
            pltpu.VMEM((256,), jnp.int32),
            pltpu.VMEM((16,), jnp.int32),
        ],
        compiler_params=pltpu.CompilerParams(needs_layout_passes=False),
    )
    zs = [z_half(et[k * gh:(k + 1) * gh], nt[k * gh:(k + 1) * gh])
          for k in range(8)]
    hs = [sel(z) for z in zs]
    ht = jnp.concatenate(hs, axis=0)
    return ht.T
